# baseline (speedup 1.0000x reference)
import functools

import jax
import jax.numpy as jnp
from jax import lax
from jax.experimental import pallas as pl
from jax.experimental.pallas import tpu as pltpu
from jax.experimental.pallas import tpu_sc as plsc

N = 10000
E = 320000
D = 128
H = 32
C = 2

NC = 2
NS = 16
L = 16
NW = NC * NS
NPAD = 10240
K = 128
CPW = 80
EPW = K * CPW
EPAD = EPW * NW
NCHUNK = EPAD // K
RPT = NPAD // NS
NSLOT = 10
DEPTH = 5

_mesh = plsc.VectorSubcoreMesh(
    core_axis_name="c", subcore_axis_name="s", num_cores=NC, num_subcores=NS
)
_sc_params = pltpu.CompilerParams(
    use_tc_tiling_on_sc=False, needs_layout_passes=False
)


@functools.partial(
    pl.kernel,
    out_type=jax.ShapeDtypeStruct((2, NW, NPAD), jnp.float32),
    mesh=_mesh,
    scratch_types=[
        pltpu.VMEM((EPW,), jnp.int32),
        pltpu.VMEM((NPAD,), jnp.float32),
    ],
    compiler_params=_sc_params,
)
def _deg_kernel(src_hbm, dst_hbm, out_hbm, idx_v, hist_v):
    wid = lax.axis_index("s") * NC + lax.axis_index("c")
    zeros = jnp.zeros((L,), jnp.float32)
    ones = jnp.ones((L,), jnp.float32)

    def histo(edge_hbm, slot):
        def zero_body(i, _):
            b = i * 4 * L
            for u in range(4):
                hist_v[pl.ds(b + u * L, L)] = zeros
            return 0

        lax.fori_loop(0, NPAD // (4 * L), zero_body, 0)
        pltpu.sync_copy(edge_hbm.at[pl.ds(wid * EPW, EPW)], idx_v)

        def add_body(i, _):
            b = i * 4 * L
            for u in range(4):
                idx = idx_v[pl.ds(b + u * L, L)]
                plsc.addupdate_scatter(hist_v, [idx], ones)
            return 0

        lax.fori_loop(0, EPW // (4 * L), add_body, 0)
        pltpu.sync_copy(hist_v, out_hbm.at[slot, wid])

    histo(src_hbm, 0)
    histo(dst_hbm, 1)


@functools.partial(
    pl.kernel,
    out_type=jax.ShapeDtypeStruct((NC, NPAD, H), jnp.float32),
    mesh=_mesh,
    scratch_types=[
        pltpu.VMEM_SHARED((NPAD, H), jnp.float32),
        pltpu.VMEM((CPW, K), jnp.int32),
        pltpu.VMEM((CPW, K), jnp.int32),
        [pltpu.VMEM((K, H), jnp.float32)] * NSLOT,
        pltpu.VMEM((RPT, H), jnp.float32),
        [pltpu.SemaphoreType.DMA] * NSLOT,
        [pltpu.SemaphoreType.DMA] * NSLOT,
    ],
    compiler_params=_sc_params,
)
def _agg_kernel(h_hbm, src_hbm, dst_hbm, out_hbm, agg_sh, src_v, dst_v,
                rows, stage_v, sem_g, sem_s):
    c = lax.axis_index("c")
    s = lax.axis_index("s")
    wid = s * NC + c
    zeros = jnp.zeros((L,), jnp.float32)

    def zero_body(i, _):
        b = i * 4
        for u in range(4):
            stage_v[b + u, pl.ds(0, L)] = zeros
            stage_v[b + u, pl.ds(L, L)] = zeros
        return 0

    lax.fori_loop(0, RPT // 4, zero_body, 0)
    pltpu.sync_copy(stage_v, agg_sh.at[pl.ds(s * RPT, RPT)])
    pltpu.sync_copy(src_hbm.at[pl.ds(wid * CPW, CPW)], src_v)
    pltpu.sync_copy(dst_hbm.at[pl.ds(wid * CPW, CPW)], dst_v)
    plsc.subcore_barrier()

    for u in range(DEPTH):
        pltpu.async_copy(h_hbm.at[src_v.at[u]], rows[u], sem_g[u])

    def body(k, _):
        for u in range(NSLOT):
            j = NSLOT * k + u
            pltpu.make_async_copy(h_hbm.at[src_v.at[j]], rows[u],
                                  sem_g[u]).wait()
            pltpu.async_copy(rows[u], agg_sh.at[dst_v.at[j]], sem_s[u],
                             add=True)
            v = (u + DEPTH) % NSLOT
            jn = j + DEPTH

            @pl.when(jn < CPW)
            def _():
                @pl.when(jn >= NSLOT)
                def _():
                    pltpu.make_async_copy(
                        rows[v], agg_sh.at[dst_v.at[jn - NSLOT]],
                        sem_s[v]).wait()

                pltpu.async_copy(h_hbm.at[src_v.at[jn]], rows[v], sem_g[v])
        return 0

    lax.fori_loop(0, CPW // NSLOT, body, 0)
    for u in range(NSLOT):
        j = CPW - NSLOT + u
        pltpu.make_async_copy(rows[u if (j % NSLOT) == u else (j % NSLOT)],
                              agg_sh.at[dst_v.at[j]],
                              sem_s[j % NSLOT]).wait()
    plsc.subcore_barrier()
    pltpu.sync_copy(agg_sh.at[pl.ds(s * RPT, RPT)], stage_v)
    pltpu.sync_copy(stage_v, out_hbm.at[c, pl.ds(s * RPT, RPT)])


def _mm_body(x_ref, w1_ref, xw_ref):
    xw_ref[...] = jnp.dot(x_ref[...], w1_ref[...],
                          preferred_element_type=jnp.float32)


_mm = pl.pallas_call(
    _mm_body,
    out_shape=jax.ShapeDtypeStruct((N, H), jnp.float32),
)


def _prep_body(deg_ref, xw_ref, scales_ref, h1_ref):
    deg = jnp.sum(deg_ref[...], axis=1)
    sc = lax.rsqrt(jnp.maximum(deg, 1.0))
    sc_t = sc.T[:N]
    scales_ref[...] = sc_t
    h1_ref[pl.ds(0, N)] = xw_ref[...] * sc_t[:, 0:1]
    h1_ref[pl.ds(N, NPAD - N)] = jnp.zeros((NPAD - N, H), jnp.float32)


_prep = pl.pallas_call(
    _prep_body,
    out_shape=(
        jax.ShapeDtypeStruct((N, 2), jnp.float32),
        jax.ShapeDtypeStruct((NPAD, H), jnp.float32),
    ),
)


def _post1_body(parts_ref, scales_ref, b1_ref, w2_ref, h2_ref):
    agg = (parts_ref[0] + parts_ref[1])[:N]
    scn = scales_ref[...]
    t = jnp.maximum(agg * scn[:, 1:2] + b1_ref[...], 0.0)
    h2_ref[pl.ds(0, N)] = jnp.dot(t * scn[:, 0:1], w2_ref[...],
                                  preferred_element_type=jnp.float32)
    h2_ref[pl.ds(N, NPAD - N)] = jnp.zeros((NPAD - N, H), jnp.float32)


_post1 = pl.pallas_call(
    _post1_body,
    out_shape=jax.ShapeDtypeStruct((NPAD, H), jnp.float32),
)


def _post2_body(parts_ref, scales_ref, b2_ref, mw1_ref, mb1_ref, mw2_ref,
                mb2_ref, out_ref):
    agg = (parts_ref[0] + parts_ref[1])[:N]
    scn = scales_ref[...]
    t = jnp.maximum(agg * scn[:, 1:2] + b2_ref[...], 0.0)
    m = jnp.maximum(
        jnp.dot(t, mw1_ref[...], preferred_element_type=jnp.float32)
        + mb1_ref[...], 0.0)
    out_ref[...] = (jnp.dot(m, mw2_ref[...], preferred_element_type=jnp.float32)
                    + mb2_ref[...])


_post2 = pl.pallas_call(
    _post2_body,
    out_shape=jax.ShapeDtypeStruct((N, C), jnp.float32),
)


def kernel(edge_index, x, W1, b1, W2, b2, mW1, mb1, mW2, mb2):
    pad = jnp.full((EPAD - E,), N, jnp.int32)
    src = jnp.concatenate([edge_index[0].astype(jnp.int32), pad])
    dst = jnp.concatenate([edge_index[1].astype(jnp.int32), pad])
    src2d = src.reshape(NCHUNK, K)
    dst2d = dst.reshape(NCHUNK, K)

    deg_parts = _deg_kernel(src, dst)
    xw = _mm(x, W1)
    scales, h1 = _prep(deg_parts, xw)
    parts1 = _agg_kernel(h1, src2d, dst2d)
    h2 = _post1(parts1, scales, b1, W2)
    parts2 = _agg_kernel(h2, src2d, dst2d)
    out = _post2(parts2, scales, b2, mW1, mb1, mW2, mb2)
    return out


# speedup vs baseline: 1.0160x; 1.0160x over previous
import functools

import jax
import jax.numpy as jnp
from jax import lax
from jax.experimental import pallas as pl
from jax.experimental.pallas import tpu as pltpu
from jax.experimental.pallas import tpu_sc as plsc

N = 10000
E = 320000
D = 128
H = 32
C = 2

NC = 2
NS = 16
L = 16
NW = NC * NS
NPAD = 10240
K = 128
CPW = 80
EPW = K * CPW
EPAD = EPW * NW
NCHUNK = EPAD // K
RPT = NPAD // NS
NSLOT = 10
DEPTH = 5
NCA = 1
CPWA = EPAD // K // (NCA * NS)

_mesh = plsc.VectorSubcoreMesh(
    core_axis_name="c", subcore_axis_name="s", num_cores=NC, num_subcores=NS
)
_mesh1 = plsc.VectorSubcoreMesh(
    core_axis_name="c", subcore_axis_name="s", num_cores=NCA, num_subcores=NS
)
_sc_params = pltpu.CompilerParams(
    use_tc_tiling_on_sc=False, needs_layout_passes=False
)


@functools.partial(
    pl.kernel,
    out_type=jax.ShapeDtypeStruct((2, NW, NPAD), jnp.float32),
    mesh=_mesh,
    scratch_types=[
        pltpu.VMEM((EPW,), jnp.int32),
        pltpu.VMEM((NPAD,), jnp.float32),
    ],
    compiler_params=_sc_params,
)
def _deg_kernel(src_hbm, dst_hbm, out_hbm, idx_v, hist_v):
    wid = lax.axis_index("s") * NC + lax.axis_index("c")
    zeros = jnp.zeros((L,), jnp.float32)
    ones = jnp.ones((L,), jnp.float32)

    def histo(edge_hbm, slot):
        def zero_body(i, _):
            b = i * 4 * L
            for u in range(4):
                hist_v[pl.ds(b + u * L, L)] = zeros
            return 0

        lax.fori_loop(0, NPAD // (4 * L), zero_body, 0)
        pltpu.sync_copy(edge_hbm.at[pl.ds(wid * EPW, EPW)], idx_v)

        def add_body(i, _):
            b = i * 4 * L
            for u in range(4):
                idx = idx_v[pl.ds(b + u * L, L)]
                plsc.addupdate_scatter(hist_v, [idx], ones)
            return 0

        lax.fori_loop(0, EPW // (4 * L), add_body, 0)
        pltpu.sync_copy(hist_v, out_hbm.at[slot, wid])

    histo(src_hbm, 0)
    histo(dst_hbm, 1)


@functools.partial(
    pl.kernel,
    out_type=jax.ShapeDtypeStruct((NCA, NPAD, H), jnp.float32),
    mesh=_mesh1,
    scratch_types=[
        pltpu.VMEM_SHARED((NPAD, H), jnp.float32),
        pltpu.VMEM((CPWA, K), jnp.int32),
        pltpu.VMEM((CPWA, K), jnp.int32),
        [pltpu.VMEM((K, H), jnp.float32)] * NSLOT,
        pltpu.VMEM((RPT, H), jnp.float32),
        [pltpu.SemaphoreType.DMA] * NSLOT,
        [pltpu.SemaphoreType.DMA] * NSLOT,
    ],
    compiler_params=_sc_params,
)
def _agg_kernel(h_hbm, src_hbm, dst_hbm, out_hbm, agg_sh, src_v, dst_v,
                rows, stage_v, sem_g, sem_s):
    c = lax.axis_index("c")
    s = lax.axis_index("s")
    wid = s * NCA + c
    zeros = jnp.zeros((L,), jnp.float32)

    def zero_body(i, _):
        b = i * 4
        for u in range(4):
            stage_v[b + u, pl.ds(0, L)] = zeros
            stage_v[b + u, pl.ds(L, L)] = zeros
        return 0

    lax.fori_loop(0, RPT // 4, zero_body, 0)
    pltpu.sync_copy(stage_v, agg_sh.at[pl.ds(s * RPT, RPT)])
    pltpu.sync_copy(src_hbm.at[pl.ds(wid * CPWA, CPWA)], src_v)
    pltpu.sync_copy(dst_hbm.at[pl.ds(wid * CPWA, CPWA)], dst_v)
    plsc.subcore_barrier()

    for u in range(DEPTH):
        pltpu.async_copy(h_hbm.at[src_v.at[u]], rows[u], sem_g[u])

    def body(k, _):
        for u in range(NSLOT):
            j = NSLOT * k + u
            pltpu.make_async_copy(h_hbm.at[src_v.at[j]], rows[u],
                                  sem_g[u]).wait()
            pltpu.async_copy(rows[u], agg_sh.at[dst_v.at[j]], sem_s[u],
                             add=True)
            v = (u + DEPTH) % NSLOT
            jn = j + DEPTH

            @pl.when(jn < CPWA)
            def _():
                @pl.when(jn >= NSLOT)
                def _():
                    pltpu.make_async_copy(
                        rows[v], agg_sh.at[dst_v.at[jn - NSLOT]],
                        sem_s[v]).wait()

                pltpu.async_copy(h_hbm.at[src_v.at[jn]], rows[v], sem_g[v])
        return 0

    lax.fori_loop(0, CPWA // NSLOT, body, 0)
    for u in range(NSLOT):
        j = CPWA - NSLOT + u
        pltpu.make_async_copy(rows[u if (j % NSLOT) == u else (j % NSLOT)],
                              agg_sh.at[dst_v.at[j]],
                              sem_s[j % NSLOT]).wait()
    plsc.subcore_barrier()
    pltpu.sync_copy(agg_sh.at[pl.ds(s * RPT, RPT)], stage_v)
    pltpu.sync_copy(stage_v, out_hbm.at[c, pl.ds(s * RPT, RPT)])


def _mm_body(x_ref, w1_ref, xw_ref):
    xw_ref[...] = jnp.dot(x_ref[...], w1_ref[...],
                          preferred_element_type=jnp.float32)


_mm = pl.pallas_call(
    _mm_body,
    out_shape=jax.ShapeDtypeStruct((N, H), jnp.float32),
)


def _prep_body(deg_ref, xw_ref, scales_ref, h1_ref):
    deg = jnp.sum(deg_ref[...], axis=1)
    sc = lax.rsqrt(jnp.maximum(deg, 1.0))
    sc_t = sc.T[:N]
    scales_ref[...] = sc_t
    h1_ref[pl.ds(0, N)] = xw_ref[...] * sc_t[:, 0:1]
    h1_ref[pl.ds(N, NPAD - N)] = jnp.zeros((NPAD - N, H), jnp.float32)


_prep = pl.pallas_call(
    _prep_body,
    out_shape=(
        jax.ShapeDtypeStruct((N, 2), jnp.float32),
        jax.ShapeDtypeStruct((NPAD, H), jnp.float32),
    ),
)


def _post1_body(parts_ref, scales_ref, b1_ref, w2_ref, h2_ref):
    agg = parts_ref[0][:N]
    scn = scales_ref[...]
    t = jnp.maximum(agg * scn[:, 1:2] + b1_ref[...], 0.0)
    h2_ref[pl.ds(0, N)] = jnp.dot(t * scn[:, 0:1], w2_ref[...],
                                  preferred_element_type=jnp.float32)
    h2_ref[pl.ds(N, NPAD - N)] = jnp.zeros((NPAD - N, H), jnp.float32)


_post1 = pl.pallas_call(
    _post1_body,
    out_shape=jax.ShapeDtypeStruct((NPAD, H), jnp.float32),
)


def _post2_body(parts_ref, scales_ref, b2_ref, mw1_ref, mb1_ref, mw2_ref,
                mb2_ref, out_ref):
    agg = parts_ref[0][:N]
    scn = scales_ref[...]
    t = jnp.maximum(agg * scn[:, 1:2] + b2_ref[...], 0.0)
    m = jnp.maximum(
        jnp.dot(t, mw1_ref[...], preferred_element_type=jnp.float32)
        + mb1_ref[...], 0.0)
    out_ref[...] = (jnp.dot(m, mw2_ref[...], preferred_element_type=jnp.float32)
                    + mb2_ref[...])


_post2 = pl.pallas_call(
    _post2_body,
    out_shape=jax.ShapeDtypeStruct((N, C), jnp.float32),
)


def kernel(edge_index, x, W1, b1, W2, b2, mW1, mb1, mW2, mb2):
    pad = jnp.full((EPAD - E,), N, jnp.int32)
    src = jnp.concatenate([edge_index[0].astype(jnp.int32), pad])
    dst = jnp.concatenate([edge_index[1].astype(jnp.int32), pad])
    src2d = src.reshape(NCHUNK, K)
    dst2d = dst.reshape(NCHUNK, K)

    deg_parts = _deg_kernel(src, dst)
    xw = _mm(x, W1)
    scales, h1 = _prep(deg_parts, xw)
    parts1 = _agg_kernel(h1, src2d, dst2d)
    h2 = _post1(parts1, scales, b1, W2)
    parts2 = _agg_kernel(h2, src2d, dst2d)
    out = _post2(parts2, scales, b2, mW1, mb1, mW2, mb2)
    return out


# speedup vs baseline: 1.0661x; 1.0493x over previous
import functools

import jax
import jax.numpy as jnp
from jax import lax
from jax.experimental import pallas as pl
from jax.experimental.pallas import tpu as pltpu
from jax.experimental.pallas import tpu_sc as plsc

N = 10000
E = 320000
D = 128
H = 32
C = 2

NC = 2
NS = 16
L = 16
NW = NC * NS
NPAD = 10240
K = 128
CPW = 80
EPW = K * CPW
EPAD = EPW * NW
NCHUNK = EPAD // K
RPT = NPAD // NS
NSLOT = 10
DEPTH = 5
NCA = 1
CPWA = EPAD // K // (NCA * NS)

_mesh = plsc.VectorSubcoreMesh(
    core_axis_name="c", subcore_axis_name="s", num_cores=NC, num_subcores=NS
)
_mesh1 = plsc.VectorSubcoreMesh(
    core_axis_name="c", subcore_axis_name="s", num_cores=NCA, num_subcores=NS
)
_sc_params = pltpu.CompilerParams(
    use_tc_tiling_on_sc=False, needs_layout_passes=False
)


@functools.partial(
    pl.kernel,
    out_type=jax.ShapeDtypeStruct((2, NW, NPAD), jnp.float32),
    mesh=_mesh,
    scratch_types=[
        pltpu.VMEM((EPW,), jnp.int32),
        pltpu.VMEM((NPAD,), jnp.float32),
        pltpu.VMEM((NPAD,), jnp.float32),
    ],
    compiler_params=_sc_params,
)
def _deg_kernel(ed_hbm, out_hbm, idx_v, hs_v, hd_v):
    wid = lax.axis_index("s") * NC + lax.axis_index("c")
    zeros = jnp.zeros((L,), jnp.float32)
    ones = jnp.ones((L,), jnp.float32)
    mask16 = jnp.full((L,), 0xFFFF, jnp.int32)

    def zero_body(i, _):
        b = i * 4 * L
        for u in range(4):
            hs_v[pl.ds(b + u * L, L)] = zeros
            hd_v[pl.ds(b + u * L, L)] = zeros
        return 0

    lax.fori_loop(0, NPAD // (4 * L), zero_body, 0)
    pltpu.sync_copy(ed_hbm.at[pl.ds(wid * EPW, EPW)], idx_v)

    def add_body(i, _):
        b = i * 2 * L
        for u in range(2):
            w = idx_v[pl.ds(b + u * L, L)]
            plsc.addupdate_scatter(hs_v, [jnp.bitwise_and(w, mask16)], ones)
            plsc.addupdate_scatter(hd_v, [lax.shift_right_logical(w, 16)],
                                   ones)
        return 0

    lax.fori_loop(0, EPW // (2 * L), add_body, 0)
    pltpu.sync_copy(hs_v, out_hbm.at[0, wid])
    pltpu.sync_copy(hd_v, out_hbm.at[1, wid])


@functools.partial(
    pl.kernel,
    out_type=jax.ShapeDtypeStruct((NCA, NPAD, H), jnp.float32),
    mesh=_mesh1,
    scratch_types=[
        pltpu.VMEM_SHARED((NPAD, H), jnp.float32),
        pltpu.VMEM((CPWA, K), jnp.int32),
        pltpu.VMEM((CPWA, K), jnp.int32),
        [pltpu.VMEM((K, H), jnp.float32)] * NSLOT,
        pltpu.VMEM((RPT, H), jnp.float32),
        [pltpu.SemaphoreType.DMA] * NSLOT,
        [pltpu.SemaphoreType.DMA] * NSLOT,
    ],
    compiler_params=_sc_params,
)
def _agg_kernel(h_hbm, ed_hbm, out_hbm, agg_sh, src_v, dst_v,
                rows, stage_v, sem_g, sem_s):
    c = lax.axis_index("c")
    s = lax.axis_index("s")
    wid = s * NCA + c
    zeros = jnp.zeros((L,), jnp.float32)

    def zero_body(i, _):
        b = i * 4
        for u in range(4):
            stage_v[b + u, pl.ds(0, L)] = zeros
            stage_v[b + u, pl.ds(L, L)] = zeros
        return 0

    lax.fori_loop(0, RPT // 4, zero_body, 0)
    pltpu.sync_copy(stage_v, agg_sh.at[pl.ds(s * RPT, RPT)])
    pltpu.sync_copy(ed_hbm.at[pl.ds(wid * CPWA, CPWA)], src_v)
    mask16 = jnp.full((L,), 0xFFFF, jnp.int32)

    def unpack_body(i, _):
        b = i * 2 * L
        for u in range(2):
            r = (b + u * L) // K
            o = (b + u * L) % K
            w = src_v[r, pl.ds(o, L)]
            dst_v[r, pl.ds(o, L)] = lax.shift_right_logical(w, 16)
            src_v[r, pl.ds(o, L)] = jnp.bitwise_and(w, mask16)
        return 0

    lax.fori_loop(0, CPWA * K // (2 * L), unpack_body, 0)
    plsc.subcore_barrier()

    for u in range(DEPTH):
        pltpu.async_copy(h_hbm.at[src_v.at[u]], rows[u], sem_g[u])

    def body(k, _):
        for u in range(NSLOT):
            j = NSLOT * k + u
            pltpu.make_async_copy(h_hbm.at[src_v.at[j]], rows[u],
                                  sem_g[u]).wait()
            pltpu.async_copy(rows[u], agg_sh.at[dst_v.at[j]], sem_s[u],
                             add=True)
            v = (u + DEPTH) % NSLOT
            jn = j + DEPTH

            @pl.when(jn < CPWA)
            def _():
                @pl.when(jn >= NSLOT)
                def _():
                    pltpu.make_async_copy(
                        rows[v], agg_sh.at[dst_v.at[jn - NSLOT]],
                        sem_s[v]).wait()

                pltpu.async_copy(h_hbm.at[src_v.at[jn]], rows[v], sem_g[v])
        return 0

    lax.fori_loop(0, CPWA // NSLOT, body, 0)
    for u in range(NSLOT):
        j = CPWA - NSLOT + u
        pltpu.make_async_copy(rows[u if (j % NSLOT) == u else (j % NSLOT)],
                              agg_sh.at[dst_v.at[j]],
                              sem_s[j % NSLOT]).wait()
    plsc.subcore_barrier()
    pltpu.sync_copy(agg_sh.at[pl.ds(s * RPT, RPT)], stage_v)
    pltpu.sync_copy(stage_v, out_hbm.at[c, pl.ds(s * RPT, RPT)])


def _mm_body(x_ref, w1_ref, xw_ref):
    xw_ref[...] = jnp.dot(x_ref[...], w1_ref[...],
                          preferred_element_type=jnp.float32)


_mm = pl.pallas_call(
    _mm_body,
    out_shape=jax.ShapeDtypeStruct((N, H), jnp.float32),
)


def _prep_body(deg_ref, xw_ref, scales_ref, h1_ref):
    deg = jnp.sum(deg_ref[...], axis=1)
    sc = lax.rsqrt(jnp.maximum(deg, 1.0))
    sc_t = sc.T[:N]
    scales_ref[...] = sc_t
    h1_ref[pl.ds(0, N)] = xw_ref[...] * sc_t[:, 0:1]
    h1_ref[pl.ds(N, NPAD - N)] = jnp.zeros((NPAD - N, H), jnp.float32)


_prep = pl.pallas_call(
    _prep_body,
    out_shape=(
        jax.ShapeDtypeStruct((N, 2), jnp.float32),
        jax.ShapeDtypeStruct((NPAD, H), jnp.float32),
    ),
)


def _post1_body(parts_ref, scales_ref, b1_ref, w2_ref, h2_ref):
    agg = parts_ref[0][:N]
    scn = scales_ref[...]
    t = jnp.maximum(agg * scn[:, 1:2] + b1_ref[...], 0.0)
    h2_ref[pl.ds(0, N)] = jnp.dot(t * scn[:, 0:1], w2_ref[...],
                                  preferred_element_type=jnp.float32)
    h2_ref[pl.ds(N, NPAD - N)] = jnp.zeros((NPAD - N, H), jnp.float32)


_post1 = pl.pallas_call(
    _post1_body,
    out_shape=jax.ShapeDtypeStruct((NPAD, H), jnp.float32),
)


def _post2_body(parts_ref, scales_ref, b2_ref, mw1_ref, mb1_ref, mw2_ref,
                mb2_ref, out_ref):
    agg = parts_ref[0][:N]
    scn = scales_ref[...]
    t = jnp.maximum(agg * scn[:, 1:2] + b2_ref[...], 0.0)
    m = jnp.maximum(
        jnp.dot(t, mw1_ref[...], preferred_element_type=jnp.float32)
        + mb1_ref[...], 0.0)
    out_ref[...] = (jnp.dot(m, mw2_ref[...], preferred_element_type=jnp.float32)
                    + mb2_ref[...])


_post2 = pl.pallas_call(
    _post2_body,
    out_shape=jax.ShapeDtypeStruct((N, C), jnp.float32),
)


def kernel(edge_index, x, W1, b1, W2, b2, mW1, mb1, mW2, mb2):
    pad = jnp.full((EPAD - E,), N, jnp.int32)
    src = jnp.concatenate([edge_index[0].astype(jnp.int32), pad])
    dst = jnp.concatenate([edge_index[1].astype(jnp.int32), pad])
    packed = jnp.bitwise_or(src, dst << 16)
    packed2d = packed.reshape(NCHUNK, K)

    deg_parts = _deg_kernel(packed)
    xw = _mm(x, W1)
    scales, h1 = _prep(deg_parts, xw)
    parts1 = _agg_kernel(h1, packed2d)
    h2 = _post1(parts1, scales, b1, W2)
    parts2 = _agg_kernel(h2, packed2d)
    out = _post2(parts2, scales, b2, mW1, mb1, mW2, mb2)
    return out


# speedup vs baseline: 1.1636x; 1.0914x over previous
import functools

import jax
import jax.numpy as jnp
from jax import lax
from jax.experimental import pallas as pl
from jax.experimental.pallas import tpu as pltpu
from jax.experimental.pallas import tpu_sc as plsc

N = 10000
E = 320000
D = 128
H = 32
C = 2

NC = 2
NS = 16
L = 16
NW = NC * NS
NPAD = 10240
K = 128
CPW = 80
EPW = K * CPW
EPAD = EPW * NW
NCHUNK = EPAD // K
RPT = NPAD // NS
NSLOT = 8
DEPTH = 4
NCA = 1
CPWA = EPAD // K // (NCA * NS)

_mesh = plsc.VectorSubcoreMesh(
    core_axis_name="c", subcore_axis_name="s", num_cores=NC, num_subcores=NS
)
_mesh1 = plsc.VectorSubcoreMesh(
    core_axis_name="c", subcore_axis_name="s", num_cores=NCA, num_subcores=NS
)
_sc_params = pltpu.CompilerParams(
    use_tc_tiling_on_sc=False, needs_layout_passes=False
)


@functools.partial(
    pl.kernel,
    out_type=jax.ShapeDtypeStruct((2, NW, NPAD), jnp.float32),
    mesh=_mesh,
    scratch_types=[
        pltpu.VMEM((EPW,), jnp.int32),
        pltpu.VMEM((NPAD,), jnp.float32),
        pltpu.VMEM((NPAD,), jnp.float32),
    ],
    compiler_params=_sc_params,
)
def _deg_kernel(ed_hbm, out_hbm, idx_v, hs_v, hd_v):
    wid = lax.axis_index("s") * NC + lax.axis_index("c")
    zeros = jnp.zeros((L,), jnp.float32)
    ones = jnp.ones((L,), jnp.float32)
    mask16 = jnp.full((L,), 0xFFFF, jnp.int32)

    def zero_body(i, _):
        b = i * 4 * L
        for u in range(4):
            hs_v[pl.ds(b + u * L, L)] = zeros
            hd_v[pl.ds(b + u * L, L)] = zeros
        return 0

    lax.fori_loop(0, NPAD // (4 * L), zero_body, 0)
    pltpu.sync_copy(ed_hbm.at[pl.ds(wid * EPW, EPW)], idx_v)

    def add_body(i, _):
        b = i * 2 * L
        for u in range(2):
            w = idx_v[pl.ds(b + u * L, L)]
            plsc.addupdate_scatter(hs_v, [jnp.bitwise_and(w, mask16)], ones)
            plsc.addupdate_scatter(hd_v, [lax.shift_right_logical(w, 16)],
                                   ones)
        return 0

    lax.fori_loop(0, EPW // (2 * L), add_body, 0)
    pltpu.sync_copy(hs_v, out_hbm.at[0, wid])
    pltpu.sync_copy(hd_v, out_hbm.at[1, wid])


@functools.partial(
    pl.kernel,
    out_type=jax.ShapeDtypeStruct((NCA, NPAD, H), jnp.float32),
    mesh=_mesh1,
    scratch_types=[
        pltpu.VMEM_SHARED((NPAD, H), jnp.float32),
        pltpu.VMEM((CPWA, K), jnp.int32),
        pltpu.VMEM((CPWA, K), jnp.int32),
        [pltpu.VMEM((K, H // 2), jnp.int32)] * NSLOT,
        [pltpu.VMEM((K, H), jnp.float32)] * NSLOT,
        pltpu.VMEM((RPT, H), jnp.float32),
        [pltpu.SemaphoreType.DMA] * NSLOT,
        [pltpu.SemaphoreType.DMA] * NSLOT,
    ],
    compiler_params=_sc_params,
)
def _agg_kernel(h_hbm, ed_hbm, out_hbm, agg_sh, src_v, dst_v,
                rows, rowsf, stage_v, sem_g, sem_s):
    c = lax.axis_index("c")
    s = lax.axis_index("s")
    wid = s * NCA + c
    zeros = jnp.zeros((L,), jnp.float32)

    def zero_body(i, _):
        b = i * 4
        for u in range(4):
            stage_v[b + u, pl.ds(0, L)] = zeros
            stage_v[b + u, pl.ds(L, L)] = zeros
        return 0

    lax.fori_loop(0, RPT // 4, zero_body, 0)
    pltpu.sync_copy(stage_v, agg_sh.at[pl.ds(s * RPT, RPT)])
    pltpu.sync_copy(ed_hbm.at[pl.ds(wid * CPWA, CPWA)], src_v)
    mask16 = jnp.full((L,), 0xFFFF, jnp.int32)

    def unpack_body(i, _):
        b = i * 2 * L
        for u in range(2):
            r = (b + u * L) // K
            o = (b + u * L) % K
            w = src_v[r, pl.ds(o, L)]
            dst_v[r, pl.ds(o, L)] = lax.shift_right_logical(w, 16)
            src_v[r, pl.ds(o, L)] = jnp.bitwise_and(w, mask16)
        return 0

    lax.fori_loop(0, CPWA * K // (2 * L), unpack_body, 0)
    plsc.subcore_barrier()

    for u in range(DEPTH):
        pltpu.async_copy(h_hbm.at[src_v.at[u]], rows[u], sem_g[u])

    himask = jnp.full((L,), -65536, jnp.int32)

    def body(k, _):
        for u in range(NSLOT):
            j = NSLOT * k + u
            pltpu.make_async_copy(h_hbm.at[src_v.at[j]], rows[u],
                                  sem_g[u]).wait()

            def widen(r, _):
                w = rows[u][r]
                lo = plsc.bitcast(jnp.left_shift(w, 16), jnp.float32)
                hi = plsc.bitcast(jnp.bitwise_and(w, himask), jnp.float32)
                rowsf[u][r, pl.ds(0, L)] = lo
                rowsf[u][r, pl.ds(L, L)] = hi
                return 0

            lax.fori_loop(0, K, widen, 0)
            pltpu.async_copy(rowsf[u], agg_sh.at[dst_v.at[j]], sem_s[u],
                             add=True)
            v = (u + DEPTH) % NSLOT
            jn = j + DEPTH

            @pl.when(jn < CPWA)
            def _():
                @pl.when(jn >= NSLOT)
                def _():
                    pltpu.make_async_copy(
                        rowsf[v], agg_sh.at[dst_v.at[jn - NSLOT]],
                        sem_s[v]).wait()

                pltpu.async_copy(h_hbm.at[src_v.at[jn]], rows[v], sem_g[v])
        return 0

    lax.fori_loop(0, CPWA // NSLOT, body, 0)
    for u in range(NSLOT):
        j = CPWA - NSLOT + u
        pltpu.make_async_copy(rows[u if (j % NSLOT) == u else (j % NSLOT)],
                              agg_sh.at[dst_v.at[j]],
                              sem_s[j % NSLOT]).wait()
    plsc.subcore_barrier()
    pltpu.sync_copy(agg_sh.at[pl.ds(s * RPT, RPT)], stage_v)
    pltpu.sync_copy(stage_v, out_hbm.at[c, pl.ds(s * RPT, RPT)])


def _pack_rows(t):
    a = lax.bitcast_convert_type(t[:, :16].astype(jnp.bfloat16), jnp.uint16)
    b = lax.bitcast_convert_type(t[:, 16:].astype(jnp.bfloat16), jnp.uint16)
    w = (b.astype(jnp.uint32) << 16) | a.astype(jnp.uint32)
    return lax.bitcast_convert_type(w, jnp.int32)


def _mm_body(x_ref, w1_ref, xw_ref):
    xw_ref[...] = jnp.dot(x_ref[...], w1_ref[...],
                          preferred_element_type=jnp.float32)


_mm = pl.pallas_call(
    _mm_body,
    out_shape=jax.ShapeDtypeStruct((N, H), jnp.float32),
)


def _prep_body(deg_ref, xw_ref, scales_ref, h1_ref):
    deg = jnp.sum(deg_ref[...], axis=1)
    sc = lax.rsqrt(jnp.maximum(deg, 1.0))
    sc_t = sc.T[:N]
    scales_ref[...] = sc_t
    t = xw_ref[...] * sc_t[:, 0:1]
    h1_ref[pl.ds(0, N)] = _pack_rows(t)
    h1_ref[pl.ds(N, NPAD - N)] = jnp.zeros((NPAD - N, H // 2), jnp.int32)


_prep = pl.pallas_call(
    _prep_body,
    out_shape=(
        jax.ShapeDtypeStruct((N, 2), jnp.float32),
        jax.ShapeDtypeStruct((NPAD, H // 2), jnp.int32),
    ),
)


def _post1_body(parts_ref, scales_ref, b1_ref, w2_ref, h2_ref):
    agg = parts_ref[0][:N]
    scn = scales_ref[...]
    t = jnp.maximum(agg * scn[:, 1:2] + b1_ref[...], 0.0)
    t2 = jnp.dot(t * scn[:, 0:1], w2_ref[...],
                 preferred_element_type=jnp.float32)
    h2_ref[pl.ds(0, N)] = _pack_rows(t2)
    h2_ref[pl.ds(N, NPAD - N)] = jnp.zeros((NPAD - N, H // 2), jnp.int32)


_post1 = pl.pallas_call(
    _post1_body,
    out_shape=jax.ShapeDtypeStruct((NPAD, H // 2), jnp.int32),
)


def _post2_body(parts_ref, scales_ref, b2_ref, mw1_ref, mb1_ref, mw2_ref,
                mb2_ref, out_ref):
    agg = parts_ref[0][:N]
    scn = scales_ref[...]
    t = jnp.maximum(agg * scn[:, 1:2] + b2_ref[...], 0.0)
    m = jnp.maximum(
        jnp.dot(t, mw1_ref[...], preferred_element_type=jnp.float32)
        + mb1_ref[...], 0.0)
    out_ref[...] = (jnp.dot(m, mw2_ref[...], preferred_element_type=jnp.float32)
                    + mb2_ref[...])


_post2 = pl.pallas_call(
    _post2_body,
    out_shape=jax.ShapeDtypeStruct((N, C), jnp.float32),
)


def kernel(edge_index, x, W1, b1, W2, b2, mW1, mb1, mW2, mb2):
    pad = jnp.full((EPAD - E,), N, jnp.int32)
    src = jnp.concatenate([edge_index[0].astype(jnp.int32), pad])
    dst = jnp.concatenate([edge_index[1].astype(jnp.int32), pad])
    packed = jnp.bitwise_or(src, dst << 16)
    packed2d = packed.reshape(NCHUNK, K)

    deg_parts = _deg_kernel(packed)
    xw = _mm(x, W1)
    scales, h1 = _prep(deg_parts, xw)
    parts1 = _agg_kernel(h1, packed2d)
    h2 = _post1(parts1, scales, b1, W2)
    parts2 = _agg_kernel(h2, packed2d)
    out = _post2(parts2, scales, b2, mW1, mb1, mW2, mb2)
    return out
